# speedup vs baseline: 1.4290x; 1.4290x over previous
import functools

import jax
import jax.numpy as jnp
from jax.experimental import pallas as pl
from jax.experimental.pallas import tpu as pltpu

M = 10000
D = 512
B = 1024
H = 8
DH = D // H

MB = 1000
NM = M // MB
SCALE = 1.0 / (DH ** 0.5)
NEG = -1e30


def _attn_body(mem_ref, query_ref, wq_ref, wk_ref, wv_ref, wo_ref, out_ref,
               q_s, acc_s, m_s, l_s):
    j = pl.program_id(0)

    @pl.when(j == 0)
    def _init():
        qb = query_ref[...].astype(jnp.bfloat16)
        wq = wq_ref[...].astype(jnp.bfloat16)
        q = jax.lax.dot_general(qb, wq, (((1,), (0,)), ((), ())),
                                preferred_element_type=jnp.float32)
        q = q * SCALE
        for h in range(H):
            q_s[h] = q[:, h * DH:(h + 1) * DH].astype(jnp.bfloat16)
        m_s[...] = jnp.full((H, B), NEG, jnp.float32)
        l_s[...] = jnp.zeros((H, B), jnp.float32)

    mb = mem_ref[...].astype(jnp.bfloat16)
    wk = wk_ref[...].astype(jnp.bfloat16)
    wv = wv_ref[...].astype(jnp.bfloat16)
    k = jax.lax.dot_general(mb, wk, (((1,), (0,)), ((), ())),
                            preferred_element_type=jnp.float32).astype(jnp.bfloat16)
    v = jax.lax.dot_general(mb, wv, (((1,), (0,)), ((), ())),
                            preferred_element_type=jnp.float32).astype(jnp.bfloat16)

    for h in range(H):
        qh = q_s[h]
        kh = k[:, h * DH:(h + 1) * DH]
        vh = v[:, h * DH:(h + 1) * DH]
        s = jax.lax.dot_general(qh, kh, (((1,), (1,)), ((), ())),
                                preferred_element_type=jnp.float32)
        m_prev = m_s[h]
        m_new = jnp.maximum(m_prev, jnp.max(s, axis=1))
        alpha = jnp.exp(m_prev - m_new)
        p = jnp.exp(s - m_new[:, None])
        l_s[h] = l_s[h] * alpha + jnp.sum(p, axis=1)
        m_s[h] = m_new
        pv = jax.lax.dot_general(p.astype(jnp.bfloat16), vh,
                                 (((1,), (0,)), ((), ())),
                                 preferred_element_type=jnp.float32)
        prev = jnp.where(j == 0, jnp.zeros_like(pv), acc_s[h])
        acc_s[h] = prev * alpha[:, None] + pv

    @pl.when(j == NM - 1)
    def _final():
        wo = wo_ref[...].astype(jnp.bfloat16)
        out = jnp.zeros((B, D), jnp.float32)
        for h in range(H):
            ctx = (acc_s[h] / l_s[h][:, None]).astype(jnp.bfloat16)
            out = out + jax.lax.dot_general(
                ctx, wo[h * DH:(h + 1) * DH, :], (((1,), (0,)), ((), ())),
                preferred_element_type=jnp.float32)
        out_ref[...] = out


def _attention(mem2, query, Wq, Wk, Wv, Wo, interpret=False):
    return pl.pallas_call(
        _attn_body,
        grid=(NM,),
        in_specs=[
            pl.BlockSpec((MB, D), lambda j: (j, 0)),
            pl.BlockSpec((B, D), lambda j: (0, 0)),
            pl.BlockSpec((D, D), lambda j: (0, 0)),
            pl.BlockSpec((D, D), lambda j: (0, 0)),
            pl.BlockSpec((D, D), lambda j: (0, 0)),
            pl.BlockSpec((D, D), lambda j: (0, 0)),
        ],
        out_specs=pl.BlockSpec((B, D), lambda j: (0, 0)),
        out_shape=jax.ShapeDtypeStruct((B, D), jnp.float32),
        scratch_shapes=[
            pltpu.VMEM((H, B, DH), jnp.bfloat16),
            pltpu.VMEM((H, B, DH), jnp.float32),
            pltpu.VMEM((H, B), jnp.float32),
            pltpu.VMEM((H, B), jnp.float32),
        ],
        compiler_params=pltpu.CompilerParams(
            dimension_semantics=("arbitrary",),
        ),
        interpret=interpret,
    )(mem2, query, Wq, Wk, Wv, Wo)


def kernel(mem, idx, val, query, Wq, Wk, Wv, Wo):
    mem2 = mem.at[idx].set(val)
    return _attention(mem2, query, Wq, Wk, Wv, Wo)
